# Initial kernel scaffold; baseline (speedup 1.0000x reference)
import jax
import jax.numpy as jnp
from jax.experimental import pallas as pl


def kernel(inputs, targets):
    raise NotImplementedError("write your pallas kernel here")


# speedup vs baseline: 16.5687x; 16.5687x over previous
import jax
import jax.numpy as jnp
from jax.experimental import pallas as pl
from jax.experimental.pallas import tpu as pltpu

_N = 8 * 512 * 512
_K = int(0.7 * _N)
_R, _C = 2048, 1024
_BR = 256
_NB = _R // _BR


def _select_body(x_ref, y_ref, s_ref, cnt_ref, t_ref, bits_ref):
    def loss_blk(i, c):
        x = x_ref[pl.ds(i * _BR, _BR), :]
        y = y_ref[pl.ds(i * _BR, _BR), :]
        l = jnp.maximum(x, 0.0) - x * y + jnp.log1p(jnp.exp(-jnp.abs(x)))
        bits_ref[pl.ds(i * _BR, _BR), :] = jax.lax.bitcast_convert_type(
            l, jnp.int32)
        return c

    jax.lax.fori_loop(0, _NB, loss_blk, 0)

    def count_gt(t):
        def blk(j, acc):
            b = bits_ref[pl.ds(j * _BR, _BR), :]
            return acc + jnp.sum((b > t).astype(jnp.int32))

        return jax.lax.fori_loop(0, _NB, blk, jnp.int32(0))

    def bisect(_, carry):
        lo, hi = carry
        mid = lo + (hi - lo) // 2
        c = count_gt(mid)
        pred = c < _K
        return (jnp.where(pred, lo, mid + 1), jnp.where(pred, mid, hi))

    lo, hi = jax.lax.fori_loop(
        0, 31, bisect, (jnp.int32(0), jnp.int32(0x7F800000)))
    tstar = hi

    def fin(j, carry):
        s, c = carry
        b = bits_ref[pl.ds(j * _BR, _BR), :]
        m = b > tstar
        v = jax.lax.bitcast_convert_type(b, jnp.float32)
        return (s + jnp.sum(jnp.where(m, v, 0.0)),
                c + jnp.sum(m.astype(jnp.int32)))

    s, c = jax.lax.fori_loop(0, _NB, fin, (jnp.float32(0.0), jnp.int32(0)))
    s_ref[0, 0] = s
    cnt_ref[0, 0] = c
    t_ref[0, 0] = tstar


def kernel(inputs, targets):
    x = inputs.reshape(_R, _C)
    y = targets.reshape(_R, _C)
    s, c, t = pl.pallas_call(
        _select_body,
        out_shape=[
            jax.ShapeDtypeStruct((1, 1), jnp.float32),
            jax.ShapeDtypeStruct((1, 1), jnp.int32),
            jax.ShapeDtypeStruct((1, 1), jnp.int32),
        ],
        out_specs=[
            pl.BlockSpec(memory_space=pltpu.SMEM),
            pl.BlockSpec(memory_space=pltpu.SMEM),
            pl.BlockSpec(memory_space=pltpu.SMEM),
        ],
        scratch_shapes=[pltpu.VMEM((_R, _C), jnp.int32)],
    )(x, y)
    tau = jax.lax.bitcast_convert_type(t[0, 0], jnp.float32)
    k = jnp.float32(_K)
    return (s[0, 0] + (k - c[0, 0].astype(jnp.float32)) * tau) / k
